# Initial kernel scaffold; baseline (speedup 1.0000x reference)
import jax
import jax.numpy as jnp
from jax.experimental import pallas as pl


def kernel(x, edge_index, gcn_W1, gcn_b1, gcn_W2, gcn_b2, sage_Wl1, sage_bl1, sage_Wr1, sage_Wl2, sage_bl2, sage_Wr2, gcn_gamma, gcn_beta, sage_gamma, sage_beta, fusion_W, fusion_b):
    raise NotImplementedError("write your pallas kernel here")


# speedup vs baseline: 5.1753x; 5.1753x over previous
import functools

import jax
import jax.numpy as jnp
from jax import lax
from jax.experimental import pallas as pl
from jax.experimental.pallas import tpu as pltpu
from jax.experimental.pallas import tpu_sc as plsc

N = 10000
E = 320000
D = 128
NC = 2
NS = 16
NW = NC * NS
CHUNK = 128
CPW = 80
NCHP = NW * CPW
EP = NCHP * CHUNK
DW = 16
NP = 10240
RPS = NP // NS
TRASH = N

_f32 = jnp.float32


def _sc_agg_body(m_hbm, src_hbm, dst_hbm, zer_hbm, out_hbm,
                 src_v, dst_v, rows_v, sem, acc_sh):
    cid = lax.axis_index("c")
    sid = lax.axis_index("s")
    wid = sid * NC + cid
    base = sid * RPS
    pltpu.sync_copy(zer_hbm.at[pl.ds(base, RPS)], acc_sh.at[pl.ds(base, RPS)])
    pltpu.sync_copy(src_hbm.at[pl.ds(wid * CPW, CPW)], src_v)
    pltpu.sync_copy(dst_hbm.at[pl.ds(wid * CPW, CPW)], dst_v)
    plsc.subcore_barrier()

    def body(j, _):
        pltpu.async_copy(m_hbm.at[src_v.at[j]], rows_v, sem).wait()
        pltpu.sync_copy(rows_v, acc_sh.at[dst_v.at[j]], add=True)
        return 0

    lax.fori_loop(0, CPW, body, 0)
    plsc.subcore_barrier()
    pltpu.sync_copy(acc_sh.at[pl.ds(base, RPS)],
                    out_hbm.at[cid, pl.ds(base, RPS)])


def _sc_deg_body(dst_hbm, zerd_hbm, oned_hbm, deg_hbm,
                 dst_v, one_v, sem, dacc_sh):
    cid = lax.axis_index("c")
    sid = lax.axis_index("s")
    wid = sid * NC + cid
    base = sid * RPS
    pltpu.sync_copy(zerd_hbm.at[pl.ds(base, RPS)],
                    dacc_sh.at[pl.ds(base, RPS)])
    pltpu.sync_copy(oned_hbm, one_v)
    pltpu.sync_copy(dst_hbm.at[pl.ds(wid * CPW, CPW)], dst_v)
    plsc.subcore_barrier()

    def body(j, _):
        pltpu.sync_copy(one_v, dacc_sh.at[dst_v.at[j]], add=True)
        return 0

    lax.fori_loop(0, CPW, body, 0)
    plsc.subcore_barrier()
    pltpu.sync_copy(dacc_sh.at[pl.ds(base, RPS)],
                    deg_hbm.at[cid, pl.ds(base, RPS)])


_SC_MESH = dict(core_axis_name="c", subcore_axis_name="s")

_sc_deg = pl.kernel(
    _sc_deg_body,
    out_type=jax.ShapeDtypeStruct((NC, NP, DW), _f32),
    mesh=plsc.VectorSubcoreMesh(**_SC_MESH),
    scratch_types=[
        pltpu.VMEM((CPW, CHUNK), jnp.int32),
        pltpu.VMEM((CHUNK, DW), _f32),
        pltpu.SemaphoreType.DMA,
        pltpu.VMEM_SHARED((NP, DW), _f32),
    ],
)

_sc_agg = pl.kernel(
    _sc_agg_body,
    out_type=jax.ShapeDtypeStruct((NC, NP, D), _f32),
    mesh=plsc.VectorSubcoreMesh(**_SC_MESH),
    scratch_types=[
        pltpu.VMEM((CPW, CHUNK), jnp.int32),
        pltpu.VMEM((CPW, CHUNK), jnp.int32),
        pltpu.VMEM((CHUNK, D), _f32),
        pltpu.SemaphoreType.DMA,
        pltpu.VMEM_SHARED((NP, D), _f32),
    ],
)


_B = 512
_GRID = NP // _B


def _deg_stats(degp_ref):
    deg = degp_ref[0, :, 0:1] + degp_ref[1, :, 0:1] + 1.0
    dis = lax.rsqrt(deg)
    invc = 1.0 / jnp.maximum(deg - 1.0, 1.0)
    return dis, invc


def _tc1_body(x_ref, s1p_ref, degp_ref, w1_ref, wl1_ref, bl1_ref, wr1_ref,
              mg1_ref, s1_ref):
    dis, invc = _deg_stats(degp_ref)
    x = x_ref[...]
    h1 = jnp.dot(x, w1_ref[...], preferred_element_type=_f32)
    mg1_ref[...] = dis * h1
    mean1 = (s1p_ref[0] + s1p_ref[1]) * invc
    sage1 = (jnp.dot(mean1, wl1_ref[...], preferred_element_type=_f32)
             + bl1_ref[...]
             + jnp.dot(x, wr1_ref[...], preferred_element_type=_f32))
    s1_ref[...] = jnp.maximum(sage1, 0.0)


def _tc2_body(mg1_ref, g1p_ref, s2p_ref, degp_ref, s1_ref,
              w2_ref, wl2_ref, bl2_ref, wr2_ref, b1_ref,
              mg2_ref, sage2_ref):
    dis, invc = _deg_stats(degp_ref)
    gcn1 = dis * (g1p_ref[0] + g1p_ref[1]) + dis * mg1_ref[...] + b1_ref[...]
    g1 = jnp.maximum(gcn1, 0.0)
    h2 = jnp.dot(g1, w2_ref[...], preferred_element_type=_f32)
    mg2_ref[...] = dis * h2
    mean2 = (s2p_ref[0] + s2p_ref[1]) * invc
    sage2_ref[...] = (jnp.dot(mean2, wl2_ref[...], preferred_element_type=_f32)
                      + bl2_ref[...]
                      + jnp.dot(s1_ref[...], wr2_ref[...],
                                preferred_element_type=_f32))


def _layer_norm(v, g, b):
    mu = jnp.mean(v, axis=-1, keepdims=True)
    var = jnp.mean((v - mu) * (v - mu), axis=-1, keepdims=True)
    return (v - mu) * lax.rsqrt(var + 1e-5) * g + b


def _tc3_body(mg2_ref, g2p_ref, degp_ref, sage2_ref,
              b2_ref, gg_ref, gb_ref, sg_ref, sb_ref, fw_ref, fb_ref,
              out_ref):
    dis, _ = _deg_stats(degp_ref)
    gcn2 = dis * (g2p_ref[0] + g2p_ref[1]) + dis * mg2_ref[...] + b2_ref[...]
    gl = _layer_norm(gcn2, gg_ref[...], gb_ref[...])
    sl = _layer_norm(sage2_ref[...], sg_ref[...], sb_ref[...])
    out_ref[...] = (jnp.dot(gl, fw_ref[0], preferred_element_type=_f32)
                    + jnp.dot(sl, fw_ref[1], preferred_element_type=_f32)
                    + fb_ref[...])


def _row_spec(width=D):
    return pl.BlockSpec((_B, width), lambda i: (i, 0))


def _part_spec(width=D):
    return pl.BlockSpec((NC, _B, width), lambda i: (0, i, 0))


def _w_spec():
    return pl.BlockSpec((D, D), lambda i: (0, 0))


def _v_spec():
    return pl.BlockSpec((1, D), lambda i: (0, 0))


_tc1 = pl.pallas_call(
    _tc1_body,
    grid=(_GRID,),
    in_specs=[_row_spec(), _part_spec(), _part_spec(DW),
              _w_spec(), _w_spec(), _v_spec(), _w_spec()],
    out_specs=[_row_spec(), _row_spec()],
    out_shape=[jax.ShapeDtypeStruct((N, D), _f32),
               jax.ShapeDtypeStruct((N, D), _f32)],
)

_tc2 = pl.pallas_call(
    _tc2_body,
    grid=(_GRID,),
    in_specs=[_row_spec(), _part_spec(), _part_spec(), _part_spec(DW),
              _row_spec(), _w_spec(), _w_spec(), _v_spec(), _w_spec(),
              _v_spec()],
    out_specs=[_row_spec(), _row_spec()],
    out_shape=[jax.ShapeDtypeStruct((N, D), _f32),
               jax.ShapeDtypeStruct((N, D), _f32)],
)

_tc3 = pl.pallas_call(
    _tc3_body,
    grid=(_GRID,),
    in_specs=[_row_spec(), _part_spec(), _part_spec(DW), _row_spec(),
              _v_spec(), _v_spec(), _v_spec(), _v_spec(), _v_spec(),
              pl.BlockSpec((2, D, D), lambda i: (0, 0, 0)), _v_spec()],
    out_specs=_row_spec(),
    out_shape=jax.ShapeDtypeStruct((N, D), _f32),
)


def kernel(x, edge_index, gcn_W1, gcn_b1, gcn_W2, gcn_b2,
           sage_Wl1, sage_bl1, sage_Wr1, sage_Wl2, sage_bl2, sage_Wr2,
           gcn_gamma, gcn_beta, sage_gamma, sage_beta, fusion_W, fusion_b):
    pad = EP - E
    src = jnp.concatenate(
        [edge_index[0].astype(jnp.int32),
         jnp.zeros((pad,), jnp.int32)]).reshape(NCHP, CHUNK)
    dst = jnp.concatenate(
        [edge_index[1].astype(jnp.int32),
         jnp.full((pad,), TRASH, jnp.int32)]).reshape(NCHP, CHUNK)
    zer = jnp.zeros((NP, D), _f32)
    zerd = jnp.zeros((NP, DW), _f32)
    oned = jnp.ones((CHUNK, DW), _f32)

    degp = _sc_deg(dst, zerd, oned)
    s1p = _sc_agg(x, src, dst, zer)
    mg1, s1 = _tc1(x, s1p, degp, gcn_W1, sage_Wl1,
                   sage_bl1.reshape(1, D), sage_Wr1)
    g1p = _sc_agg(mg1, src, dst, zer)
    s2p = _sc_agg(s1, src, dst, zer)
    mg2, sage2 = _tc2(mg1, g1p, s2p, degp, s1, gcn_W2, sage_Wl2,
                      sage_bl2.reshape(1, D), sage_Wr2, gcn_b1.reshape(1, D))
    g2p = _sc_agg(mg2, src, dst, zer)
    out = _tc3(mg2, g2p, degp, sage2, gcn_b2.reshape(1, D),
               gcn_gamma.reshape(1, D), gcn_beta.reshape(1, D),
               sage_gamma.reshape(1, D), sage_beta.reshape(1, D),
               fusion_W.reshape(2, D, D), fusion_b.reshape(1, D))
    return out


# speedup vs baseline: 5.6558x; 1.0928x over previous
import functools

import jax
import jax.numpy as jnp
from jax import lax
from jax.experimental import pallas as pl
from jax.experimental.pallas import tpu as pltpu
from jax.experimental.pallas import tpu_sc as plsc

N = 10000
E = 320000
D = 128
NC = 2
NS = 16
NW = NC * NS
CHUNK = 128
CPW = 80
NCHP = NW * CPW
EP = NCHP * CHUNK
DW = 16
NP = 10240
RPS = NP // NS
TRASH = N

_f32 = jnp.float32


NB = 2
BI = 64


def _sc_agg_body(m_hbm, src_hbm, dst_hbm, zer_hbm, out_hbm,
                 src_v, dst_v, g0, g1, acc_sh):
    pl.run_scoped(
        functools.partial(_sc_agg_inner, m_hbm, src_hbm, dst_hbm, zer_hbm,
                          out_hbm, (g0, g1), acc_sh, src_v, dst_v),
        *([pltpu.VMEM((CHUNK, D), _f32)] * NB),
    )


def _sc_agg_inner(m_hbm, src_hbm, dst_hbm, zer_hbm, out_hbm,
                  gsem, acc_sh, src_v, dst_v, *rows):
    cid = lax.axis_index("c")
    sid = lax.axis_index("s")
    wid = sid * NC + cid
    base = sid * RPS
    cbase = wid * CPW
    pltpu.sync_copy(zer_hbm.at[pl.ds(base, RPS)], acc_sh.at[pl.ds(base, RPS)])
    plsc.subcore_barrier()

    def fire_gather(j, b):
        pltpu.async_copy(m_hbm.at[src_v.at[j]], rows[b], gsem[b])

    def wait_gather(j, b):
        pltpu.make_async_copy(m_hbm.at[src_v.at[j]], rows[b],
                              gsem[b]).wait()

    def phase(cstart, cnt):
        pltpu.sync_copy(src_hbm.at[pl.ds(cbase + cstart, cnt)],
                        src_v.at[pl.ds(0, cnt)])
        pltpu.sync_copy(dst_hbm.at[pl.ds(cbase + cstart, cnt)],
                        dst_v.at[pl.ds(0, cnt)])
        fire_gather(0, 0)

        def body(jj, _):
            for b in range(NB):
                j = jj * NB + b
                wait_gather(j, b)

                @pl.when(j + 1 < cnt)
                def _():
                    fire_gather(j + 1, (b + 1) % NB)

                pltpu.sync_copy(rows[b], acc_sh.at[dst_v.at[j]], add=True)
            return 0

        lax.fori_loop(0, cnt // NB, body, 0)

    phase(0, BI)
    phase(BI, CPW - BI)
    plsc.subcore_barrier()
    pltpu.sync_copy(acc_sh.at[pl.ds(base, RPS)],
                    out_hbm.at[cid, pl.ds(base, RPS)])


def _sc_deg_body(dst_hbm, zerd_hbm, oned_hbm, deg_hbm,
                 dst_v, one_v, sem, dacc_sh):
    cid = lax.axis_index("c")
    sid = lax.axis_index("s")
    wid = sid * NC + cid
    base = sid * RPS
    pltpu.sync_copy(zerd_hbm.at[pl.ds(base, RPS)],
                    dacc_sh.at[pl.ds(base, RPS)])
    pltpu.sync_copy(oned_hbm, one_v)
    pltpu.sync_copy(dst_hbm.at[pl.ds(wid * CPW, CPW)], dst_v)
    plsc.subcore_barrier()

    def body(j, _):
        pltpu.sync_copy(one_v, dacc_sh.at[dst_v.at[j]], add=True)
        return 0

    lax.fori_loop(0, CPW, body, 0)
    plsc.subcore_barrier()
    pltpu.sync_copy(dacc_sh.at[pl.ds(base, RPS)],
                    deg_hbm.at[cid, pl.ds(base, RPS)])


_SC_MESH = dict(core_axis_name="c", subcore_axis_name="s")

_sc_deg = pl.kernel(
    _sc_deg_body,
    out_type=jax.ShapeDtypeStruct((NC, NP, DW), _f32),
    mesh=plsc.VectorSubcoreMesh(**_SC_MESH),
    scratch_types=[
        pltpu.VMEM((CPW, CHUNK), jnp.int32),
        pltpu.VMEM((CHUNK, DW), _f32),
        pltpu.SemaphoreType.DMA,
        pltpu.VMEM_SHARED((NP, DW), _f32),
    ],
)

_sc_agg = pl.kernel(
    _sc_agg_body,
    out_type=jax.ShapeDtypeStruct((NC, NP, D), _f32),
    mesh=plsc.VectorSubcoreMesh(**_SC_MESH),
    scratch_types=(
        [pltpu.VMEM((BI, CHUNK), jnp.int32),
         pltpu.VMEM((BI, CHUNK), jnp.int32)]
        + [pltpu.SemaphoreType.DMA] * NB
        + [pltpu.VMEM_SHARED((NP, D), _f32)]
    ),
)


_B = 512
_GRID = NP // _B


def _deg_stats(degp_ref):
    deg = degp_ref[0, :, 0:1] + degp_ref[1, :, 0:1] + 1.0
    dis = lax.rsqrt(deg)
    invc = 1.0 / jnp.maximum(deg - 1.0, 1.0)
    return dis, invc


def _tc1_body(x_ref, s1p_ref, degp_ref, w1_ref, wl1_ref, bl1_ref, wr1_ref,
              mg1_ref, s1_ref):
    dis, invc = _deg_stats(degp_ref)
    x = x_ref[...]
    h1 = jnp.dot(x, w1_ref[...], preferred_element_type=_f32)
    mg1_ref[...] = dis * h1
    mean1 = (s1p_ref[0] + s1p_ref[1]) * invc
    sage1 = (jnp.dot(mean1, wl1_ref[...], preferred_element_type=_f32)
             + bl1_ref[...]
             + jnp.dot(x, wr1_ref[...], preferred_element_type=_f32))
    s1_ref[...] = jnp.maximum(sage1, 0.0)


def _tc2_body(mg1_ref, g1p_ref, s2p_ref, degp_ref, s1_ref,
              w2_ref, wl2_ref, bl2_ref, wr2_ref, b1_ref,
              mg2_ref, sage2_ref):
    dis, invc = _deg_stats(degp_ref)
    gcn1 = dis * (g1p_ref[0] + g1p_ref[1]) + dis * mg1_ref[...] + b1_ref[...]
    g1 = jnp.maximum(gcn1, 0.0)
    h2 = jnp.dot(g1, w2_ref[...], preferred_element_type=_f32)
    mg2_ref[...] = dis * h2
    mean2 = (s2p_ref[0] + s2p_ref[1]) * invc
    sage2_ref[...] = (jnp.dot(mean2, wl2_ref[...], preferred_element_type=_f32)
                      + bl2_ref[...]
                      + jnp.dot(s1_ref[...], wr2_ref[...],
                                preferred_element_type=_f32))


def _layer_norm(v, g, b):
    mu = jnp.mean(v, axis=-1, keepdims=True)
    var = jnp.mean((v - mu) * (v - mu), axis=-1, keepdims=True)
    return (v - mu) * lax.rsqrt(var + 1e-5) * g + b


def _tc3_body(mg2_ref, g2p_ref, degp_ref, sage2_ref,
              b2_ref, gg_ref, gb_ref, sg_ref, sb_ref, fw_ref, fb_ref,
              out_ref):
    dis, _ = _deg_stats(degp_ref)
    gcn2 = dis * (g2p_ref[0] + g2p_ref[1]) + dis * mg2_ref[...] + b2_ref[...]
    gl = _layer_norm(gcn2, gg_ref[...], gb_ref[...])
    sl = _layer_norm(sage2_ref[...], sg_ref[...], sb_ref[...])
    out_ref[...] = (jnp.dot(gl, fw_ref[0], preferred_element_type=_f32)
                    + jnp.dot(sl, fw_ref[1], preferred_element_type=_f32)
                    + fb_ref[...])


def _row_spec(width=D):
    return pl.BlockSpec((_B, width), lambda i: (i, 0))


def _part_spec(width=D):
    return pl.BlockSpec((NC, _B, width), lambda i: (0, i, 0))


def _w_spec():
    return pl.BlockSpec((D, D), lambda i: (0, 0))


def _v_spec():
    return pl.BlockSpec((1, D), lambda i: (0, 0))


_tc1 = pl.pallas_call(
    _tc1_body,
    grid=(_GRID,),
    in_specs=[_row_spec(), _part_spec(), _part_spec(DW),
              _w_spec(), _w_spec(), _v_spec(), _w_spec()],
    out_specs=[_row_spec(), _row_spec()],
    out_shape=[jax.ShapeDtypeStruct((N, D), _f32),
               jax.ShapeDtypeStruct((N, D), _f32)],
)

_tc2 = pl.pallas_call(
    _tc2_body,
    grid=(_GRID,),
    in_specs=[_row_spec(), _part_spec(), _part_spec(), _part_spec(DW),
              _row_spec(), _w_spec(), _w_spec(), _v_spec(), _w_spec(),
              _v_spec()],
    out_specs=[_row_spec(), _row_spec()],
    out_shape=[jax.ShapeDtypeStruct((N, D), _f32),
               jax.ShapeDtypeStruct((N, D), _f32)],
)

_tc3 = pl.pallas_call(
    _tc3_body,
    grid=(_GRID,),
    in_specs=[_row_spec(), _part_spec(), _part_spec(DW), _row_spec(),
              _v_spec(), _v_spec(), _v_spec(), _v_spec(), _v_spec(),
              pl.BlockSpec((2, D, D), lambda i: (0, 0, 0)), _v_spec()],
    out_specs=_row_spec(),
    out_shape=jax.ShapeDtypeStruct((N, D), _f32),
)


def kernel(x, edge_index, gcn_W1, gcn_b1, gcn_W2, gcn_b2,
           sage_Wl1, sage_bl1, sage_Wr1, sage_Wl2, sage_bl2, sage_Wr2,
           gcn_gamma, gcn_beta, sage_gamma, sage_beta, fusion_W, fusion_b):
    pad = EP - E
    src = jnp.concatenate(
        [edge_index[0].astype(jnp.int32),
         jnp.zeros((pad,), jnp.int32)]).reshape(NCHP, CHUNK)
    dst = jnp.concatenate(
        [edge_index[1].astype(jnp.int32),
         jnp.full((pad,), TRASH, jnp.int32)]).reshape(NCHP, CHUNK)
    zer = jnp.zeros((NP, D), _f32)
    zerd = jnp.zeros((NP, DW), _f32)
    oned = jnp.ones((CHUNK, DW), _f32)

    degp = _sc_deg(dst, zerd, oned)
    s1p = _sc_agg(x, src, dst, zer)
    mg1, s1 = _tc1(x, s1p, degp, gcn_W1, sage_Wl1,
                   sage_bl1.reshape(1, D), sage_Wr1)
    g1p = _sc_agg(mg1, src, dst, zer)
    s2p = _sc_agg(s1, src, dst, zer)
    mg2, sage2 = _tc2(mg1, g1p, s2p, degp, s1, gcn_W2, sage_Wl2,
                      sage_bl2.reshape(1, D), sage_Wr2, gcn_b1.reshape(1, D))
    g2p = _sc_agg(mg2, src, dst, zer)
    out = _tc3(mg2, g2p, degp, sage2, gcn_b2.reshape(1, D),
               gcn_gamma.reshape(1, D), gcn_beta.reshape(1, D),
               sage_gamma.reshape(1, D), sage_beta.reshape(1, D),
               fusion_W.reshape(2, D, D), fusion_b.reshape(1, D))
    return out
